# baseline (device time: 126846 ns/iter reference)
import functools

import jax
import jax.numpy as jnp
from jax import lax
from jax.experimental import pallas as pl
from jax.experimental.pallas import tpu as pltpu

N_DEV = 8
SQ = 1024
SKV = 1024
HQ = 8
DH = 128
D = HQ * DH
BLK = 64
N_CHUNK = 8
CHUNK = SKV // N_CHUNK
SCALE = 0.08838834764831843


def kernel(x, Wq, K_ext, V_ext, Wo):
    def body(x_ref, wq_ref, k_ref, v_ref, wo_ref, out_ref, kv_ref, ctx_ref,
             send_sems, recv_sems):
        my = lax.axis_index("i")
        left = (my - 1) % N_DEV
        right = (my + 1) % N_DEV

        barrier_sem = pltpu.get_barrier_semaphore()
        for nbr in (left, right):
            pl.semaphore_signal(barrier_sem, inc=1, device_id=(nbr,),
                                device_id_type=pl.DeviceIdType.MESH)
        pl.semaphore_wait(barrier_sem, 2)

        @pl.when(my == 0)
        def _():
            kbf = jnp.reshape(k_ref[0], (SKV, D)).astype(jnp.bfloat16)
            vbf = jnp.reshape(v_ref[0], (SKV, D)).astype(jnp.bfloat16)
            big = jnp.concatenate([kbf, vbf], axis=1)
            for c in range(N_CHUNK):
                kv_ref[c] = big[c * CHUNK:(c + 1) * CHUNK]

        sends = []
        recvs = []
        for c in range(N_CHUNK):
            sends.append(pltpu.make_async_remote_copy(
                src_ref=kv_ref.at[c], dst_ref=kv_ref.at[c],
                send_sem=send_sems.at[c], recv_sem=recv_sems.at[c],
                device_id=(right,), device_id_type=pl.DeviceIdType.MESH))
            recvs.append(pltpu.make_async_remote_copy(
                src_ref=kv_ref.at[c], dst_ref=kv_ref.at[c],
                send_sem=send_sems.at[c], recv_sem=recv_sems.at[c],
                device_id=(left,), device_id_type=pl.DeviceIdType.MESH))

        for c in range(N_CHUNK):
            @pl.when(my > 0)
            def _(c=c):
                recvs[c].wait_recv()

            @pl.when(my < N_DEV - 1)
            def _(c=c):
                sends[c].start()

        xbf = x_ref[0].astype(jnp.bfloat16)
        wqbf = wq_ref[...].astype(jnp.bfloat16)
        q = jnp.dot(xbf, wqbf, preferred_element_type=jnp.float32)
        q = (q * SCALE).astype(jnp.bfloat16)

        kvall = jnp.reshape(kv_ref[...], (SKV, 2 * D))
        rb = lax.broadcasted_iota(jnp.int32, (SQ, SKV), 0) // BLK
        cb = lax.broadcasted_iota(jnp.int32, (SQ, SKV), 1) // BLK
        mask = cb <= rb
        for h in range(HQ):
            qh = q[:, h * DH:(h + 1) * DH]
            kh = kvall[:, h * DH:(h + 1) * DH]
            vh = kvall[:, D + h * DH:D + (h + 1) * DH]
            s = lax.dot_general(qh, kh, (((1,), (1,)), ((), ())),
                                preferred_element_type=jnp.float32)
            s = jnp.where(mask, s, -1e9)
            m = jnp.max(s, axis=1, keepdims=True)
            p = jnp.exp(s - m)
            p = p / jnp.sum(p, axis=1, keepdims=True)
            ctx_h = lax.dot_general(p.astype(jnp.bfloat16), vh,
                                    (((1,), (0,)), ((), ())),
                                    preferred_element_type=jnp.float32)
            ctx_ref[:, h * DH:(h + 1) * DH] = ctx_h.astype(jnp.bfloat16)

        wobf = wo_ref[...].astype(jnp.bfloat16)
        out_ref[0] = jnp.dot(ctx_ref[...], wobf,
                             preferred_element_type=jnp.float32)

        @pl.when(my < N_DEV - 1)
        def _():
            for c in range(N_CHUNK):
                sends[c].wait_send()

        @functools.partial(pl.run_scoped,
                           exit_sem=pltpu.SemaphoreType.REGULAR)
        def _(exit_sem):
            for nbr in (left, right):
                pl.semaphore_signal(exit_sem, inc=1, device_id=(nbr,),
                                    device_id_type=pl.DeviceIdType.MESH)
            pl.semaphore_wait(exit_sem, 2)

    return pl.pallas_call(
        body,
        out_shape=jax.ShapeDtypeStruct((1, SQ, D), jnp.float32),
        in_specs=[pl.BlockSpec(memory_space=pltpu.VMEM)] * 5,
        out_specs=pl.BlockSpec(memory_space=pltpu.VMEM),
        scratch_shapes=[
            pltpu.VMEM((N_CHUNK, CHUNK, 2 * D), jnp.bfloat16),
            pltpu.VMEM((SQ, D), jnp.bfloat16),
            pltpu.SemaphoreType.DMA((N_CHUNK,)),
            pltpu.SemaphoreType.DMA((N_CHUNK,)),
        ],
        compiler_params=pltpu.CompilerParams(collective_id=0),
    )(x, Wq, K_ext, V_ext, Wo)


# device time: 67955 ns/iter; 1.8666x vs baseline; 1.8666x over previous
import functools

import jax
import jax.numpy as jnp
from jax import lax
from jax.experimental import pallas as pl
from jax.experimental.pallas import tpu as pltpu

N_DEV = 8
SQ = 1024
SKV = 1024
HQ = 8
DH = 128
D = HQ * DH
BLK = 64
N_CHUNK = 8
CHUNK = SQ // N_CHUNK
SCALE = 0.08838834764831843


def kernel(x, Wq, K_ext, V_ext, Wo):
    def body(x_ref, wq_ref, k_ref, v_ref, wo_ref, out_ref, comm_ref,
             cw_sems, ccw_sems, recv_sems):
        my = lax.axis_index("i")
        left = (my - 1) % N_DEV
        right = (my + 1) % N_DEV

        barrier_sem = pltpu.get_barrier_semaphore()
        for nbr in (left, right):
            pl.semaphore_signal(barrier_sem, inc=1, device_id=(nbr,),
                                device_id_type=pl.DeviceIdType.MESH)
        pl.semaphore_wait(barrier_sem, 2)

        recv_src = jnp.where(my <= 4, left, right)
        fwd_dst = jnp.where(my <= 3, right, left)
        forwards = ((my >= 1) & (my <= 3)) | (my >= 6)

        send_cw, send_ccw, recvs, fwds = [], [], [], []
        for c in range(N_CHUNK):
            send_cw.append(pltpu.make_async_remote_copy(
                src_ref=comm_ref.at[c], dst_ref=comm_ref.at[c],
                send_sem=cw_sems.at[c], recv_sem=recv_sems.at[c],
                device_id=(1,), device_id_type=pl.DeviceIdType.MESH))
            send_ccw.append(pltpu.make_async_remote_copy(
                src_ref=comm_ref.at[c], dst_ref=comm_ref.at[c],
                send_sem=ccw_sems.at[c], recv_sem=recv_sems.at[c],
                device_id=(N_DEV - 1,), device_id_type=pl.DeviceIdType.MESH))
            recvs.append(pltpu.make_async_remote_copy(
                src_ref=comm_ref.at[c], dst_ref=comm_ref.at[c],
                send_sem=cw_sems.at[c], recv_sem=recv_sems.at[c],
                device_id=(recv_src,), device_id_type=pl.DeviceIdType.MESH))
            fwds.append(pltpu.make_async_remote_copy(
                src_ref=comm_ref.at[c], dst_ref=comm_ref.at[c],
                send_sem=cw_sems.at[c], recv_sem=recv_sems.at[c],
                device_id=(fwd_dst,), device_id_type=pl.DeviceIdType.MESH))

        @pl.when(my == 0)
        def _():
            xbf = x_ref[0].astype(jnp.bfloat16)
            wqbf = wq_ref[...].astype(jnp.bfloat16)
            wobf = wo_ref[...].astype(jnp.bfloat16)
            q = jnp.dot(xbf, wqbf, preferred_element_type=jnp.float32)
            q = (q * SCALE).astype(jnp.bfloat16)
            kbf = jnp.reshape(k_ref[0], (SKV, D)).astype(jnp.bfloat16)
            vbf = jnp.reshape(v_ref[0], (SKV, D)).astype(jnp.bfloat16)

            for c in range(N_CHUNK):
                w = (c + 1) * CHUNK
                rb = lax.broadcasted_iota(jnp.int32, (CHUNK, w), 0) // BLK \
                    + 2 * c
                cb = lax.broadcasted_iota(jnp.int32, (CHUNK, w), 1) // BLK
                mask = cb <= rb
                ctx_parts = []
                for h in range(HQ):
                    qh = q[c * CHUNK:(c + 1) * CHUNK, h * DH:(h + 1) * DH]
                    kh = kbf[:w, h * DH:(h + 1) * DH]
                    vh = vbf[:w, h * DH:(h + 1) * DH]
                    s = lax.dot_general(qh, kh, (((1,), (1,)), ((), ())),
                                        preferred_element_type=jnp.float32)
                    s = jnp.where(mask, s, -1e9)
                    m = jnp.max(s, axis=1, keepdims=True)
                    p = jnp.exp(s - m)
                    p = p / jnp.sum(p, axis=1, keepdims=True)
                    ctx_h = lax.dot_general(p.astype(jnp.bfloat16), vh,
                                            (((1,), (0,)), ((), ())),
                                            preferred_element_type=jnp.float32)
                    ctx_parts.append(ctx_h.astype(jnp.bfloat16))
                ctx_c = jnp.concatenate(ctx_parts, axis=1)
                out_c = jnp.dot(ctx_c, wobf,
                                preferred_element_type=jnp.float32)
                out_ref[0, c * CHUNK:(c + 1) * CHUNK, :] = out_c
                comm_ref[c] = out_c.astype(jnp.bfloat16)
                send_cw[c].start()
                send_ccw[c].start()

            for c in range(N_CHUNK):
                send_cw[c].wait_send()
                send_ccw[c].wait_send()

        @pl.when(my > 0)
        def _():
            for c in range(N_CHUNK):
                recvs[c].wait_recv()

                @pl.when(forwards)
                def _(c=c):
                    fwds[c].start()

                out_ref[0, c * CHUNK:(c + 1) * CHUNK, :] = (
                    comm_ref[c].astype(jnp.float32))

            @pl.when(forwards)
            def _():
                for c in range(N_CHUNK):
                    fwds[c].wait_send()

        @functools.partial(pl.run_scoped,
                           exit_sem=pltpu.SemaphoreType.REGULAR)
        def _(exit_sem):
            for nbr in (left, right):
                pl.semaphore_signal(exit_sem, inc=1, device_id=(nbr,),
                                    device_id_type=pl.DeviceIdType.MESH)
            pl.semaphore_wait(exit_sem, 2)

    return pl.pallas_call(
        body,
        out_shape=jax.ShapeDtypeStruct((1, SQ, D), jnp.float32),
        in_specs=[pl.BlockSpec(memory_space=pltpu.VMEM)] * 5,
        out_specs=pl.BlockSpec(memory_space=pltpu.VMEM),
        scratch_shapes=[
            pltpu.VMEM((N_CHUNK, CHUNK, D), jnp.bfloat16),
            pltpu.SemaphoreType.DMA((N_CHUNK,)),
            pltpu.SemaphoreType.DMA((N_CHUNK,)),
            pltpu.SemaphoreType.DMA((N_CHUNK,)),
        ],
        compiler_params=pltpu.CompilerParams(collective_id=0),
    )(x, Wq, K_ext, V_ext, Wo)


# device time: 55463 ns/iter; 2.2870x vs baseline; 1.2252x over previous
import functools

import jax
import jax.numpy as jnp
from jax import lax
from jax.experimental import pallas as pl
from jax.experimental.pallas import tpu as pltpu

N_DEV = 8
SQ = 1024
SKV = 1024
HQ = 8
DH = 128
D = HQ * DH
BLK = 64
N_CHUNK = 8
CHUNK = SQ // N_CHUNK
SCALE = 0.08838834764831843

ROOT_CHILDREN = (1, 3, 4)
PARENT = {1: 0, 2: 1, 3: 0, 4: 0, 5: 4, 6: 2, 7: 3}
CHILDREN = {0: (1, 3, 4), 1: (2,), 2: (6,), 3: (7,), 4: (5,)}


def kernel(x, Wq, K_ext, V_ext, Wo):
    def body(x_ref, wq_ref, k_ref, v_ref, wo_ref, out_ref, comm_ref,
             root_sems, fwd_sems, recv_sems):
        my = lax.axis_index("i")

        def tree_barrier(sem):
            for d, par in PARENT.items():
                @pl.when(my == d)
                def _(par=par):
                    pl.semaphore_signal(sem, inc=1, device_id=(par,),
                                        device_id_type=pl.DeviceIdType.MESH)
            for d, kids in CHILDREN.items():
                @pl.when(my == d)
                def _(kids=kids):
                    for kid in kids:
                        pl.semaphore_signal(sem, inc=1, device_id=(kid,),
                                            device_id_type=pl.DeviceIdType.MESH)
            @pl.when(my == 0)
            def _():
                pl.semaphore_wait(sem, 3)

            @pl.when((my >= 1) & (my <= 4))
            def _():
                pl.semaphore_wait(sem, 2)

            @pl.when(my >= 5)
            def _():
                pl.semaphore_wait(sem, 1)

        tree_barrier(pltpu.get_barrier_semaphore())

        recv_src = jnp.int32(0)
        for d, par in PARENT.items():
            recv_src = jnp.where(my == d, par, recv_src)
        fwd_dst = jnp.int32(0)
        forwards = (my >= 1) & (my <= 4)
        for d, kids in CHILDREN.items():
            if d != 0:
                fwd_dst = jnp.where(my == d, kids[0], fwd_dst)

        root_sends, recvs, fwds = [], [], []
        for c in range(N_CHUNK):
            root_sends.append([pltpu.make_async_remote_copy(
                src_ref=comm_ref.at[c], dst_ref=comm_ref.at[c],
                send_sem=root_sems.at[j, c], recv_sem=recv_sems.at[c],
                device_id=(kid,), device_id_type=pl.DeviceIdType.MESH)
                for j, kid in enumerate(ROOT_CHILDREN)])
            recvs.append(pltpu.make_async_remote_copy(
                src_ref=comm_ref.at[c], dst_ref=comm_ref.at[c],
                send_sem=fwd_sems.at[c], recv_sem=recv_sems.at[c],
                device_id=(recv_src,), device_id_type=pl.DeviceIdType.MESH))
            fwds.append(pltpu.make_async_remote_copy(
                src_ref=comm_ref.at[c], dst_ref=comm_ref.at[c],
                send_sem=fwd_sems.at[c], recv_sem=recv_sems.at[c],
                device_id=(fwd_dst,), device_id_type=pl.DeviceIdType.MESH))

        @pl.when(my == 0)
        def _():
            wqbf = wq_ref[...].astype(jnp.bfloat16)
            wobf = wo_ref[...].astype(jnp.bfloat16)
            kbf = jnp.reshape(k_ref[0], (SKV, D)).astype(jnp.bfloat16)
            vbf = jnp.reshape(v_ref[0], (SKV, D)).astype(jnp.bfloat16)

            for c in range(N_CHUNK):
                w = (c + 1) * CHUNK
                xc = x_ref[0, c * CHUNK:(c + 1) * CHUNK, :].astype(jnp.bfloat16)
                qc = jnp.dot(xc, wqbf, preferred_element_type=jnp.float32)
                qc = (qc * SCALE).astype(jnp.bfloat16)
                rb = lax.broadcasted_iota(jnp.int32, (CHUNK, w), 0) // BLK \
                    + 2 * c
                cb = lax.broadcasted_iota(jnp.int32, (CHUNK, w), 1) // BLK
                mask = cb <= rb
                ctx_parts = []
                for h in range(HQ):
                    qh = qc[:, h * DH:(h + 1) * DH]
                    kh = kbf[:w, h * DH:(h + 1) * DH]
                    vh = vbf[:w, h * DH:(h + 1) * DH]
                    s = lax.dot_general(qh, kh, (((1,), (1,)), ((), ())),
                                        preferred_element_type=jnp.float32)
                    s = jnp.where(mask, s, -1e9)
                    m = jnp.max(s, axis=1, keepdims=True)
                    p = jnp.exp(s - m)
                    denom = jnp.sum(p, axis=1, keepdims=True)
                    ctx_h = lax.dot_general(p.astype(jnp.bfloat16), vh,
                                            (((1,), (0,)), ((), ())),
                                            preferred_element_type=jnp.float32)
                    ctx_h = ctx_h * (1.0 / denom)
                    ctx_parts.append(ctx_h.astype(jnp.bfloat16))
                ctx_c = jnp.concatenate(ctx_parts, axis=1)
                out_c = jnp.dot(ctx_c, wobf,
                                preferred_element_type=jnp.float32)
                out_ref[0, c * CHUNK:(c + 1) * CHUNK, :] = out_c
                comm_ref[c] = out_c.astype(jnp.bfloat16)
                for j in range(len(ROOT_CHILDREN)):
                    root_sends[c][j].start()

            for c in range(N_CHUNK):
                for j in range(len(ROOT_CHILDREN)):
                    root_sends[c][j].wait_send()

        @pl.when(my > 0)
        def _():
            for c in range(N_CHUNK):
                recvs[c].wait_recv()

                @pl.when(forwards)
                def _(c=c):
                    fwds[c].start()

                out_ref[0, c * CHUNK:(c + 1) * CHUNK, :] = (
                    comm_ref[c].astype(jnp.float32))

            @pl.when(forwards)
            def _():
                for c in range(N_CHUNK):
                    fwds[c].wait_send()

        @functools.partial(pl.run_scoped,
                           exit_sem=pltpu.SemaphoreType.REGULAR)
        def _(exit_sem):
            tree_barrier(exit_sem)

    return pl.pallas_call(
        body,
        out_shape=jax.ShapeDtypeStruct((1, SQ, D), jnp.float32),
        in_specs=[pl.BlockSpec(memory_space=pltpu.VMEM)] * 5,
        out_specs=pl.BlockSpec(memory_space=pltpu.VMEM),
        scratch_shapes=[
            pltpu.VMEM((N_CHUNK, CHUNK, D), jnp.bfloat16),
            pltpu.SemaphoreType.DMA((len(ROOT_CHILDREN), N_CHUNK)),
            pltpu.SemaphoreType.DMA((N_CHUNK,)),
            pltpu.SemaphoreType.DMA((N_CHUNK,)),
        ],
        compiler_params=pltpu.CompilerParams(collective_id=0),
    )(x, Wq, K_ext, V_ext, Wo)
